# Initial kernel scaffold; baseline (speedup 1.0000x reference)
import jax
import jax.numpy as jnp
from jax.experimental import pallas as pl


def kernel(x, W, b, threshold):
    raise NotImplementedError("write your pallas kernel here")


# speedup vs baseline: 2.5788x; 2.5788x over previous
import functools

import jax
import jax.numpy as jnp
from jax.experimental import pallas as pl
from jax.experimental.pallas import tpu as pltpu

K = 64


def _encode_topk_kernel(x_ref, w_ref, b_ref, th_ref, sparse_ref, idx_ref,
                        f_scr, idx_scr, *, n_hb, bh, br, hidden):
    h = pl.program_id(1)

    @pl.when(h < n_hb)
    def _matmul():
        pre = jax.lax.dot_general(
            x_ref[...], w_ref[...], (((1,), (1,)), ((), ())),
            preferred_element_type=jnp.float32)
        pre = pre + b_ref[...]
        feat = pre * (pre > th_ref[...]).astype(jnp.float32)
        f_scr[:, pl.ds(h * bh, bh)] = feat

    @pl.when(h == n_hb - 1)
    def _extract():
        cols = [jax.lax.broadcasted_iota(jnp.int32, (br, bh), 1) + blk * bh
                for blk in range(n_hb)]

        def body(i, _):
            m = jnp.full((br,), -1.0, jnp.float32)
            for blk in range(n_hb):
                m = jnp.maximum(m, jnp.max(f_scr[:, pl.ds(blk * bh, bh)],
                                           axis=1))
            idx = jnp.full((br,), hidden, jnp.int32)
            for blk in range(n_hb):
                f = f_scr[:, pl.ds(blk * bh, bh)]
                cand = jnp.where(f == m[:, None], cols[blk], hidden)
                idx = jnp.minimum(idx, jnp.min(cand, axis=1))
            idx_scr[pl.ds(i, 1), :] = idx[None, :]
            for blk in range(n_hb):
                sl = pl.ds(blk * bh, bh)
                f = f_scr[:, sl]
                f_scr[:, sl] = jnp.where(cols[blk] == idx[:, None],
                                         -(f + 1.0), f)
            return 0

        jax.lax.fori_loop(0, K, body, 0, unroll=False)
        idx_ref[...] = idx_scr[...].T

    @pl.when(h >= n_hb)
    def _emit_sparse():
        f = f_scr[:, pl.ds((h - n_hb) * bh, bh)]
        sparse_ref[...] = jnp.where(f < 0.0, -f - 1.0, 0.0)


def _encode_topk(x2d, W, b2d, th2d, *, br, bh):
    rows, in_dim = x2d.shape
    hidden = W.shape[0]
    n_rb = rows // br
    n_hb = hidden // bh
    grid = (n_rb, 2 * n_hb)
    out_shapes = (
        jax.ShapeDtypeStruct((rows, hidden), jnp.float32),
        jax.ShapeDtypeStruct((rows, K), jnp.int32),
    )
    kern = functools.partial(_encode_topk_kernel, n_hb=n_hb, bh=bh,
                             br=br, hidden=hidden)
    hmax = n_hb - 1
    return pl.pallas_call(
        kern,
        grid=grid,
        in_specs=[
            pl.BlockSpec((br, in_dim), lambda r, h: (r, 0)),
            pl.BlockSpec((bh, in_dim), lambda r, h: (jnp.minimum(h, hmax), 0)),
            pl.BlockSpec((1, bh), lambda r, h: (0, jnp.minimum(h, hmax))),
            pl.BlockSpec((1, bh), lambda r, h: (0, jnp.minimum(h, hmax))),
        ],
        out_specs=(
            pl.BlockSpec((br, bh),
                         lambda r, h: (r, jnp.maximum(h - n_hb, 0))),
            pl.BlockSpec((br, K), lambda r, h: (r, 0)),
        ),
        out_shape=out_shapes,
        scratch_shapes=[
            pltpu.VMEM((br, hidden), jnp.float32),
            pltpu.VMEM((K, br), jnp.int32),
        ],
        compiler_params=pltpu.CompilerParams(
            dimension_semantics=("arbitrary", "arbitrary")),
    )(x2d, W, b2d, th2d)


def kernel(x, W, b, threshold):
    batch, seq, in_dim = x.shape
    hidden = W.shape[0]
    rows = batch * seq
    x2d = x.reshape(rows, in_dim)
    b2d = b.reshape(1, hidden)
    th2d = threshold.reshape(1, hidden)
    br = 256 if rows % 256 == 0 else rows
    bh = 1024 if hidden % 1024 == 0 else hidden
    sparse, idx = _encode_topk(x2d, W, b2d, th2d, br=br, bh=bh)
    return (sparse.reshape(batch, seq, hidden),
            idx.reshape(batch, seq, K))


# speedup vs baseline: 7.5278x; 2.9191x over previous
import functools

import jax
import jax.numpy as jnp
from jax import lax
from jax.experimental import pallas as pl
from jax.experimental.pallas import tpu as pltpu
from jax.experimental.pallas import tpu_sc as plsc

K = 64
CHUNK = 16
CAPC = 256
CAPP = 512


def _mm_kernel(x_ref, w_ref, b_ref, th_ref, f_ref, m_ref, lo_ref, m_scr,
               mprev_scr, *, n_hb, bh, br, nchunks):
    h = pl.program_id(1)
    nct = bh // CHUNK

    pre = jax.lax.dot_general(
        x_ref[...], w_ref[...], (((1,), (1,)), ((), ())),
        preferred_element_type=jnp.float32)
    pre = pre + b_ref[...]
    feat = pre * (pre > th_ref[...]).astype(jnp.float32)
    f_ref[...] = feat
    tmax = jnp.max(feat.reshape(br, nct, CHUNK), axis=2)

    @pl.when(h % 2 == 0)
    def _stash():
        mprev_scr[...] = tmax

    @pl.when(h % 2 == 1)
    def _commit():
        m_scr[:, pl.ds((h // 2) * (2 * nct), 2 * nct)] = jnp.concatenate(
            [mprev_scr[...], tmax], axis=1)

    @pl.when(h == n_hb - 1)
    def _lo_search():
        m_int = jax.lax.bitcast_convert_type(m_scr[...], jnp.int32)

        def body(_, lohi):
            lo, hi = lohi
            mid = lo + (hi - lo) // 2
            cnt = jnp.sum((m_int >= mid[:, None]).astype(jnp.int32), axis=1)
            ok = cnt >= K
            return jnp.where(ok, mid, lo), jnp.where(ok, hi, mid)

        lo0 = jnp.zeros((br,), jnp.int32)
        hi0 = jnp.full((br,), 0x7F800000, jnp.int32)
        lo, _ = jax.lax.fori_loop(0, 31, body, (lo0, hi0))
        m_ref[...] = m_scr[...]
        lo_ref[...] = jax.lax.bitcast_convert_type(lo, jnp.float32)[:, None]


def _encode_tc(x2d, W, b2d, th2d, *, br, bh):
    rows, in_dim = x2d.shape
    hidden = W.shape[0]
    n_rb = rows // br
    n_hb = hidden // bh
    nchunks = hidden // CHUNK
    kern = functools.partial(_mm_kernel, n_hb=n_hb, bh=bh, br=br,
                             nchunks=nchunks)
    return pl.pallas_call(
        kern,
        grid=(n_rb, n_hb),
        in_specs=[
            pl.BlockSpec((br, in_dim), lambda r, h: (r, 0)),
            pl.BlockSpec((bh, in_dim), lambda r, h: (h, 0)),
            pl.BlockSpec((1, bh), lambda r, h: (0, h)),
            pl.BlockSpec((1, bh), lambda r, h: (0, h)),
        ],
        out_specs=(
            pl.BlockSpec((br, bh), lambda r, h: (r, h)),
            pl.BlockSpec((br, nchunks), lambda r, h: (r, 0)),
            pl.BlockSpec((br, 1), lambda r, h: (r, 0)),
        ),
        out_shape=(
            jax.ShapeDtypeStruct((rows, hidden), jnp.float32),
            jax.ShapeDtypeStruct((rows, nchunks), jnp.float32),
            jax.ShapeDtypeStruct((rows, 1), jnp.float32),
        ),
        scratch_shapes=[pltpu.VMEM((br, nchunks), jnp.float32),
                        pltpu.VMEM((br, bh // CHUNK), jnp.float32)],
        compiler_params=pltpu.CompilerParams(
            dimension_semantics=("arbitrary", "arbitrary")),
    )(x2d, W, b2d, th2d)


def _cmpx(av, ai, bv, bi):
    sel = av >= bv
    return (jnp.where(sel, av, bv), jnp.where(sel, ai, bi),
            jnp.where(sel, bv, av), jnp.where(sel, bi, ai))


def _rev(x):
    return jax.lax.rev(x, dimensions=(0,))


def _vsortd(v, i):
    return plsc.sort_key_val(v, i, descending=True)


def _merge16(av, ai, bv, bi):
    hv, hi_, lv, li = _cmpx(av, ai, _rev(bv), _rev(bi))
    hv, hi_ = _vsortd(hv, hi_)
    lv, li = _vsortd(lv, li)
    return hv, hi_, lv, li


def _bmerge32(h0v, h0i, h1v, h1i):
    uv, ui, wv, wi = _cmpx(h0v, h0i, h1v, h1i)
    uv, ui = _vsortd(uv, ui)
    wv, wi = _vsortd(wv, wi)
    return uv, ui, wv, wi


def _merge32(a, b):
    a0v, a0i, a1v, a1i = a
    b0v, b0i, b1v, b1i = b
    h0v, h0i, l1v, l1i = _cmpx(a0v, a0i, _rev(b1v), _rev(b1i))
    h1v, h1i, l0v, l0i = _cmpx(a1v, a1i, _rev(b0v), _rev(b0i))
    s0v, s0i, s1v, s1i = _bmerge32(h0v, h0i, h1v, h1i)
    s2v, s2i, s3v, s3i = _bmerge32(l0v, l0i, l1v, l1i)
    return ((s0v, s0i), (s1v, s1i), (s2v, s2i), (s3v, s3i))


def _sort64(vs, is_):
    s = [_vsortd(vs[k], is_[k]) for k in range(4)]
    a = _merge16(s[0][0], s[0][1], s[1][0], s[1][1])
    b = _merge16(s[2][0], s[2][1], s[3][0], s[3][1])
    return _merge32(a, b)


def _merge_top64(A, B):
    h = []
    for k in range(4):
        bv, bi = B[3 - k]
        hv, hi_, _, _ = _cmpx(A[k][0], A[k][1], _rev(bv), _rev(bi))
        h.append((hv, hi_))
    u0v, u0i, w0v, w0i = _cmpx(h[0][0], h[0][1], h[2][0], h[2][1])
    u1v, u1i, w1v, w1i = _cmpx(h[1][0], h[1][1], h[3][0], h[3][1])
    s0v, s0i, s1v, s1i = _bmerge32(u0v, u0i, u1v, u1i)
    s2v, s2i, s3v, s3i = _bmerge32(w0v, w0i, w1v, w1i)
    return ((s0v, s0i), (s1v, s1i), (s2v, s2i), (s3v, s3i))


def _sc_topk(f_g, M, lo, *, rows, hidden):
    nchunks = hidden // CHUNK
    nc, ns = 2, 16
    nw = nc * ns
    rpw = rows // nw
    mesh = plsc.VectorSubcoreMesh(core_axis_name="c", subcore_axis_name="s",
                                  num_cores=nc, num_subcores=ns)

    @functools.partial(
        pl.kernel, mesh=mesh,
        out_type=(
            jax.ShapeDtypeStruct((rows, hidden), jnp.float32),
            jax.ShapeDtypeStruct((rows, K), jnp.int32),
        ),
        scratch_types=[
            pltpu.VMEM((rpw,), jnp.float32),
            pltpu.VMEM((nchunks,), jnp.float32),
            pltpu.VMEM((CAPC + 16,), jnp.int32),
            pltpu.VMEM((CAPC + 16,), jnp.int32),
            pltpu.VMEM((CAPC, 128), jnp.float32),
            pltpu.VMEM((CAPP + 64,), jnp.float32),
            pltpu.VMEM((CAPP + 64,), jnp.int32),
            pltpu.VMEM((hidden,), jnp.float32),
            pltpu.VMEM((K,), jnp.int32),
            pltpu.SemaphoreType.DMA,
        ],
        compiler_params=pltpu.CompilerParams(needs_layout_passes=False),
    )
    def body(fg_hbm, m_hbm, lo_hbm, sp_hbm, ti_hbm,
             lo_v, m_v, cidx_v, pidx_v, gath_v, vpool, ipool, row_buf, tib,
             sem):
        wid = lax.axis_index("s") * nc + lax.axis_index("c")
        base = wid * rpw
        iota16 = jax.lax.iota(jnp.int32, 16)
        zero16 = jnp.zeros((16,), jnp.float32)

        pltpu.sync_copy(lo_hbm.at[pl.ds(base * 1, rpw)], lo_v)

        @pl.loop(0, hidden // 16)
        def _zero(i):
            row_buf[pl.ds(i * 16, 16)] = zero16

        @pl.loop(0, rpw)
        def _row(rl):
            r = base + rl
            pltpu.sync_copy(m_hbm.at[r], m_v)
            lvec = lo_v[pl.ds((rl // 16) * 16, 16)]
            lsc = jax.lax.reduce_max(
                jnp.where(iota16 == rl % 16, lvec, -1.0), axes=(0,))
            lo_spl = jnp.broadcast_to(lsc, (16,))
            pbase = r * (hidden // 128)
            pad16 = jnp.broadcast_to(pbase, (16,))

            @pl.loop(0, CAPC // 16)
            def _pad(i):
                pidx_v[pl.ds(i * 16, 16)] = pad16

            def scan_body(i, ncand):
                v = m_v[pl.ds(i * 16, 16)]
                msk = (v >= lo_spl) & (ncand < CAPC)
                ids = iota16 + i * 16
                plsc.store_compressed(cidx_v.at[pl.ds(ncand, 16)], ids,
                                      mask=msk)
                plsc.store_compressed(pidx_v.at[pl.ds(ncand, 16)],
                                      pad16 + jax.lax.shift_right_logical(
                                          ids, 3), mask=msk)
                cnt = plsc.all_reduce_population_count(msk)
                return ncand + jax.lax.reduce_max(cnt, axes=(0,))

            ncand = jax.lax.fori_loop(0, nchunks // 16, scan_body, 0,
                                      unroll=4)

            for p in range(CAPC // 128):
                @pl.when(ncand > p * 128)
                def _gather():
                    pltpu.async_copy(
                        fg_hbm.at[pidx_v.at[pl.ds(p * 128, 128)]],
                        gath_v.at[pl.ds(p * 128, 128)], sem).wait()

            def ext_body(j, np_):
                cvec = cidx_v[pl.ds((j // 16) * 16, 16)]
                csc = jax.lax.reduce_max(
                    jnp.where(iota16 == j % 16, cvec, 0), axes=(0,))
                g = gath_v[j, pl.ds((csc % 8) * 16, 16)]
                gidx = jnp.broadcast_to(csc, (16,)) * CHUNK + iota16
                msk = (g >= lo_spl) & (np_ < CAPP)
                plsc.store_compressed(vpool.at[pl.ds(np_, 16)], g, mask=msk)
                plsc.store_compressed(ipool.at[pl.ds(np_, 16)], gidx,
                                      mask=msk)
                cnt = plsc.all_reduce_population_count(msk)
                return np_ + jax.lax.reduce_max(cnt, axes=(0,))

            np_ = jax.lax.fori_loop(0, ncand, ext_body, 0)

            neg1 = jnp.full((16,), -1.0, jnp.float32)
            big = jnp.full((16,), jnp.int32(1 << 24), jnp.int32)
            for t in range(4):
                vpool[pl.ds(np_ + t * 16, 16)] = neg1
                ipool[pl.ds(np_ + t * 16, 16)] = big

            def load64(off):
                vs = [vpool[pl.ds(off + t * 16, 16)] for t in range(4)]
                is_ = [ipool[pl.ds(off + t * 16, 16)] for t in range(4)]
                return _sort64(vs, is_)

            npool = (np_ + 63) // 64

            def mrg_body(q, top):
                return _merge_top64(top, load64(q * 64))

            top = jax.lax.fori_loop(1, npool, mrg_body, load64(0))

            for t in range(4):
                tib[pl.ds(t * 16, 16)] = top[t][1]
                plsc.store_scatter(row_buf, [top[t][1]], top[t][0])
            pltpu.sync_copy(tib, ti_hbm.at[r])
            pltpu.sync_copy(row_buf, sp_hbm.at[r])
            for t in range(4):
                plsc.store_scatter(row_buf, [top[t][1]], zero16)

    return body(f_g, M, lo)


def kernel(x, W, b, threshold):
    batch, seq, in_dim = x.shape
    hidden = W.shape[0]
    rows = batch * seq
    x2d = x.reshape(rows, in_dim)
    b2d = b.reshape(1, hidden)
    th2d = threshold.reshape(1, hidden)
    F, M, lo = _encode_tc(x2d, W, b2d, th2d, br=512, bh=1024)
    f_g = F.reshape(rows * (hidden // 128), 128)
    sparse, idx = _sc_topk(f_g, M, lo.reshape(rows), rows=rows,
                           hidden=hidden)
    return (sparse.reshape(batch, seq, hidden),
            idx.reshape(batch, seq, K))


# speedup vs baseline: 8.7084x; 1.1568x over previous
import functools

import jax
import jax.numpy as jnp
from jax import lax
from jax.experimental import pallas as pl
from jax.experimental.pallas import tpu as pltpu
from jax.experimental.pallas import tpu_sc as plsc

K = 64
CHUNK = 16
CAPC = 256
CAPP = 512


def _mm_kernel(x_ref, w_ref, b_ref, th_ref, f_ref, m_ref, lo_ref, m_scr,
               mprev_scr, *, n_hb, bh, br, nchunks):
    h = pl.program_id(1)
    nct = bh // CHUNK

    pre = jax.lax.dot_general(
        x_ref[...], w_ref[...], (((1,), (1,)), ((), ())),
        preferred_element_type=jnp.float32)
    pre = pre + b_ref[...]
    feat = pre * (pre > th_ref[...]).astype(jnp.float32)
    f_ref[...] = feat
    tmax = jnp.max(feat.reshape(br, nct, CHUNK), axis=2)

    @pl.when(h % 2 == 0)
    def _stash():
        mprev_scr[...] = tmax

    @pl.when(h % 2 == 1)
    def _commit():
        m_scr[:, pl.ds((h // 2) * (2 * nct), 2 * nct)] = jnp.concatenate(
            [mprev_scr[...], tmax], axis=1)

    @pl.when(h == n_hb - 1)
    def _lo_search():
        m_int = jax.lax.bitcast_convert_type(m_scr[...], jnp.int32)

        def body(_, lohi):
            lo, hi = lohi
            mid = lo + (hi - lo) // 2
            cnt = jnp.sum((m_int >= mid[:, None]).astype(jnp.int32), axis=1)
            ok = cnt >= K
            return jnp.where(ok, mid, lo), jnp.where(ok, hi, mid)

        lo0 = jnp.zeros((br,), jnp.int32)
        hi0 = jnp.full((br,), 0x7F800000, jnp.int32)
        lo, _ = jax.lax.fori_loop(0, 31, body, (lo0, hi0))
        m_ref[...] = m_scr[...]
        lo_ref[...] = jax.lax.bitcast_convert_type(lo, jnp.float32)[:, None]


def _encode_tc(x2d, W, b2d, th2d, *, br, bh):
    rows, in_dim = x2d.shape
    hidden = W.shape[0]
    n_rb = rows // br
    n_hb = hidden // bh
    nchunks = hidden // CHUNK
    kern = functools.partial(_mm_kernel, n_hb=n_hb, bh=bh, br=br,
                             nchunks=nchunks)
    return pl.pallas_call(
        kern,
        grid=(n_rb, n_hb),
        in_specs=[
            pl.BlockSpec((br, in_dim), lambda r, h: (r, 0)),
            pl.BlockSpec((bh, in_dim), lambda r, h: (h, 0)),
            pl.BlockSpec((1, bh), lambda r, h: (0, h)),
            pl.BlockSpec((1, bh), lambda r, h: (0, h)),
        ],
        out_specs=(
            pl.BlockSpec((br, bh), lambda r, h: (r, h)),
            pl.BlockSpec((br, nchunks), lambda r, h: (r, 0)),
            pl.BlockSpec((br, 1), lambda r, h: (r, 0)),
        ),
        out_shape=(
            jax.ShapeDtypeStruct((rows, hidden), jnp.float32),
            jax.ShapeDtypeStruct((rows, nchunks), jnp.float32),
            jax.ShapeDtypeStruct((rows, 1), jnp.float32),
        ),
        scratch_shapes=[pltpu.VMEM((br, nchunks), jnp.float32),
                        pltpu.VMEM((br, bh // CHUNK), jnp.float32)],
        compiler_params=pltpu.CompilerParams(
            dimension_semantics=("arbitrary", "arbitrary")),
    )(x2d, W, b2d, th2d)


def _cmpx(av, ai, bv, bi):
    sel = av >= bv
    return (jnp.where(sel, av, bv), jnp.where(sel, ai, bi),
            jnp.where(sel, bv, av), jnp.where(sel, bi, ai))


def _rev(x):
    return jax.lax.rev(x, dimensions=(0,))


def _vsortd(v, i):
    return plsc.sort_key_val(v, i, descending=True)


def _merge16(av, ai, bv, bi):
    hv, hi_, lv, li = _cmpx(av, ai, _rev(bv), _rev(bi))
    hv, hi_ = _vsortd(hv, hi_)
    lv, li = _vsortd(lv, li)
    return hv, hi_, lv, li


def _bmerge32(h0v, h0i, h1v, h1i):
    uv, ui, wv, wi = _cmpx(h0v, h0i, h1v, h1i)
    uv, ui = _vsortd(uv, ui)
    wv, wi = _vsortd(wv, wi)
    return uv, ui, wv, wi


def _merge32(a, b):
    a0v, a0i, a1v, a1i = a
    b0v, b0i, b1v, b1i = b
    h0v, h0i, l1v, l1i = _cmpx(a0v, a0i, _rev(b1v), _rev(b1i))
    h1v, h1i, l0v, l0i = _cmpx(a1v, a1i, _rev(b0v), _rev(b0i))
    s0v, s0i, s1v, s1i = _bmerge32(h0v, h0i, h1v, h1i)
    s2v, s2i, s3v, s3i = _bmerge32(l0v, l0i, l1v, l1i)
    return ((s0v, s0i), (s1v, s1i), (s2v, s2i), (s3v, s3i))


def _sort64(vs, is_):
    s = [_vsortd(vs[k], is_[k]) for k in range(4)]
    a = _merge16(s[0][0], s[0][1], s[1][0], s[1][1])
    b = _merge16(s[2][0], s[2][1], s[3][0], s[3][1])
    return _merge32(a, b)


def _merge_top64(A, B):
    h = []
    for k in range(4):
        bv, bi = B[3 - k]
        hv, hi_, _, _ = _cmpx(A[k][0], A[k][1], _rev(bv), _rev(bi))
        h.append((hv, hi_))
    u0v, u0i, w0v, w0i = _cmpx(h[0][0], h[0][1], h[2][0], h[2][1])
    u1v, u1i, w1v, w1i = _cmpx(h[1][0], h[1][1], h[3][0], h[3][1])
    s0v, s0i, s1v, s1i = _bmerge32(u0v, u0i, u1v, u1i)
    s2v, s2i, s3v, s3i = _bmerge32(w0v, w0i, w1v, w1i)
    return ((s0v, s0i), (s1v, s1i), (s2v, s2i), (s3v, s3i))


def _sc_topk(f_g, M, lo, *, rows, hidden):
    nchunks = hidden // CHUNK
    nc, ns = 2, 16
    nw = nc * ns
    rpw = rows // nw
    mesh = plsc.VectorSubcoreMesh(core_axis_name="c", subcore_axis_name="s",
                                  num_cores=nc, num_subcores=ns)

    @functools.partial(
        pl.kernel, mesh=mesh,
        out_type=(
            jax.ShapeDtypeStruct((rows, hidden), jnp.float32),
            jax.ShapeDtypeStruct((rows, K), jnp.int32),
        ),
        scratch_types=[
            pltpu.VMEM((rpw,), jnp.float32),
            pltpu.VMEM((nchunks,), jnp.float32),
            pltpu.VMEM((CAPC + 16,), jnp.int32),
            pltpu.VMEM((CAPC + 16,), jnp.int32),
            pltpu.VMEM((CAPC, 128), jnp.float32),
            pltpu.VMEM((CAPP + 64,), jnp.float32),
            pltpu.VMEM((CAPP + 64,), jnp.int32),
            pltpu.VMEM((hidden,), jnp.float32),
            pltpu.VMEM((K,), jnp.int32),
            pltpu.SemaphoreType.DMA,
        ],
        compiler_params=pltpu.CompilerParams(needs_layout_passes=False),
    )
    def body(fg_hbm, m_hbm, lo_hbm, sp_hbm, ti_hbm,
             lo_v, m_v, cidx_v, pidx_v, gath_v, vpool, ipool, row_buf, tib,
             sem):
        wid = lax.axis_index("s") * nc + lax.axis_index("c")
        base = wid * rpw
        iota16 = jax.lax.iota(jnp.int32, 16)
        zero16 = jnp.zeros((16,), jnp.float32)

        pltpu.sync_copy(lo_hbm.at[pl.ds(base * 1, rpw)], lo_v)

        @pl.loop(0, hidden // 16)
        def _zero(i):
            row_buf[pl.ds(i * 16, 16)] = zero16

        @pl.loop(0, rpw)
        def _row(rl):
            r = base + rl
            pltpu.sync_copy(m_hbm.at[r], m_v)
            lvec = lo_v[pl.ds((rl // 16) * 16, 16)]
            lsc = jax.lax.reduce_max(
                jnp.where(iota16 == rl % 16, lvec, -1.0), axes=(0,))
            lo_spl = jnp.broadcast_to(lsc, (16,))
            pbase = r * (hidden // 128)
            pad16 = jnp.broadcast_to(pbase, (16,))

            @pl.loop(0, CAPC // 16)
            def _pad(i):
                pidx_v[pl.ds(i * 16, 16)] = pad16

            def scan_body(i, ncand):
                v = m_v[pl.ds(i * 16, 16)]
                msk = (v >= lo_spl) & (ncand < CAPC)
                ids = iota16 + i * 16
                plsc.store_compressed(cidx_v.at[pl.ds(ncand, 16)], ids,
                                      mask=msk)
                plsc.store_compressed(pidx_v.at[pl.ds(ncand, 16)],
                                      pad16 + jax.lax.shift_right_logical(
                                          ids, 3), mask=msk)
                cnt = plsc.all_reduce_population_count(msk)
                return ncand + jax.lax.reduce_max(cnt, axes=(0,))

            ncand = jax.lax.fori_loop(0, nchunks // 16, scan_body, 0,
                                      unroll=4)

            for p in range(CAPC // 128):
                @pl.when(ncand > p * 128)
                def _gather():
                    pltpu.async_copy(
                        fg_hbm.at[pidx_v.at[pl.ds(p * 128, 128)]],
                        gath_v.at[pl.ds(p * 128, 128)], sem).wait()

            def ext_body(j, np_):
                cvec = cidx_v[pl.ds((j // 16) * 16, 16)]
                csc = jax.lax.reduce_max(
                    jnp.where(iota16 == j % 16, cvec, 0), axes=(0,))
                g = gath_v[j, pl.ds((csc % 8) * 16, 16)]
                gidx = jnp.broadcast_to(csc, (16,)) * CHUNK + iota16
                msk = (g >= lo_spl) & (np_ < CAPP)
                plsc.store_compressed(vpool.at[pl.ds(np_, 16)], g, mask=msk)
                plsc.store_compressed(ipool.at[pl.ds(np_, 16)], gidx,
                                      mask=msk)
                cnt = plsc.all_reduce_population_count(msk)
                return np_ + jax.lax.reduce_max(cnt, axes=(0,))

            np_ = jax.lax.fori_loop(0, ncand, ext_body, 0)

            neg1 = jnp.full((16,), -1.0, jnp.float32)
            big = jnp.full((16,), jnp.int32(1 << 24), jnp.int32)
            for t in range(4):
                vpool[pl.ds(np_ + t * 16, 16)] = neg1
                ipool[pl.ds(np_ + t * 16, 16)] = big

            def load64(off):
                vs = [vpool[pl.ds(off + t * 16, 16)] for t in range(4)]
                is_ = [ipool[pl.ds(off + t * 16, 16)] for t in range(4)]
                return _sort64(vs, is_)

            npool = (np_ + 63) // 64

            def mrg_body(q, top):
                return _merge_top64(top, load64(q * 64))

            top = jax.lax.fori_loop(1, npool, mrg_body, load64(0))

            for t in range(4):
                tib[pl.ds(t * 16, 16)] = top[t][1]
                plsc.store_scatter(row_buf, [top[t][1]], top[t][0])
            pltpu.sync_copy(tib, ti_hbm.at[r])
            pltpu.sync_copy(row_buf, sp_hbm.at[r])
            for t in range(4):
                plsc.store_scatter(row_buf, [top[t][1]], zero16)

    return body(f_g, M, lo)


def kernel(x, W, b, threshold):
    batch, seq, in_dim = x.shape
    hidden = W.shape[0]
    rows = batch * seq
    x2d = x.reshape(rows, in_dim)
    b2d = b.reshape(1, hidden)
    th2d = threshold.reshape(1, hidden)
    nh = 2 if rows % 2 == 0 else 1
    hr = rows // nh
    sps, idxs = [], []
    for i in range(nh):
        xi = jax.lax.slice_in_dim(x2d, i * hr, (i + 1) * hr, axis=0)
        F, M, lo = _encode_tc(xi, W, b2d, th2d, br=min(512, hr), bh=1024)
        f_g = F.reshape(hr * (hidden // 128), 128)
        sp, ix = _sc_topk(f_g, M, lo.reshape(hr), rows=hr, hidden=hidden)
        sps.append(sp)
        idxs.append(ix)
    sparse = jnp.concatenate(sps, axis=0)
    idx = jnp.concatenate(idxs, axis=0)
    return (sparse.reshape(batch, seq, hidden),
            idx.reshape(batch, seq, K))


# speedup vs baseline: 9.6824x; 1.1119x over previous
import functools

import jax
import jax.numpy as jnp
from jax import lax
from jax.experimental import pallas as pl
from jax.experimental.pallas import tpu as pltpu
from jax.experimental.pallas import tpu_sc as plsc

K = 64
CHUNK = 16
CAPC = 256
CAPP = 512


def _mm_kernel(x_ref, w_ref, b_ref, th_ref, f_ref, m_ref, lo_ref, m_scr,
               mprev_scr, *, n_hb, bh, br, nchunks):
    h = pl.program_id(1)
    nct = bh // CHUNK

    pre = jax.lax.dot_general(
        x_ref[...], w_ref[...], (((1,), (1,)), ((), ())),
        preferred_element_type=jnp.float32)
    pre = pre + b_ref[...]
    feat = pre * (pre > th_ref[...]).astype(jnp.float32)
    f_ref[...] = feat
    tmax = jnp.max(feat.reshape(br, nct, CHUNK), axis=2)

    @pl.when(h % 2 == 0)
    def _stash():
        mprev_scr[...] = tmax

    @pl.when(h % 2 == 1)
    def _commit():
        m_scr[:, pl.ds((h // 2) * (2 * nct), 2 * nct)] = jnp.concatenate(
            [mprev_scr[...], tmax], axis=1)

    @pl.when(h == n_hb - 1)
    def _lo_search():
        m_int = jax.lax.bitcast_convert_type(m_scr[...], jnp.int32)

        def body(_, lohi):
            lo, hi = lohi
            mid = lo + (hi - lo) // 2
            cnt = jnp.sum((m_int >= mid[:, None]).astype(jnp.int32), axis=1)
            ok = cnt >= K
            return jnp.where(ok, mid, lo), jnp.where(ok, hi, mid)

        lo0 = jnp.zeros((br,), jnp.int32)
        hi0 = jnp.full((br,), 0x7F800000, jnp.int32)
        lo, _ = jax.lax.fori_loop(0, 31, body, (lo0, hi0))
        m_ref[...] = m_scr[...]
        lo_ref[...] = jax.lax.bitcast_convert_type(lo, jnp.float32)[:, None]


def _encode_tc(x2d, W, b2d, th2d, *, br, bh):
    rows, in_dim = x2d.shape
    hidden = W.shape[0]
    n_rb = rows // br
    n_hb = hidden // bh
    nchunks = hidden // CHUNK
    kern = functools.partial(_mm_kernel, n_hb=n_hb, bh=bh, br=br,
                             nchunks=nchunks)
    return pl.pallas_call(
        kern,
        grid=(n_rb, n_hb),
        in_specs=[
            pl.BlockSpec((br, in_dim), lambda r, h: (r, 0)),
            pl.BlockSpec((bh, in_dim), lambda r, h: (h, 0)),
            pl.BlockSpec((1, bh), lambda r, h: (0, h)),
            pl.BlockSpec((1, bh), lambda r, h: (0, h)),
        ],
        out_specs=(
            pl.BlockSpec((br, bh), lambda r, h: (r, h)),
            pl.BlockSpec((br, nchunks), lambda r, h: (r, 0)),
            pl.BlockSpec((br, 1), lambda r, h: (r, 0)),
        ),
        out_shape=(
            jax.ShapeDtypeStruct((rows, hidden), jnp.float32),
            jax.ShapeDtypeStruct((rows, nchunks), jnp.float32),
            jax.ShapeDtypeStruct((rows, 1), jnp.float32),
        ),
        scratch_shapes=[pltpu.VMEM((br, nchunks), jnp.float32),
                        pltpu.VMEM((br, bh // CHUNK), jnp.float32)],
        compiler_params=pltpu.CompilerParams(
            dimension_semantics=("arbitrary", "arbitrary")),
    )(x2d, W, b2d, th2d)


def _cmpx(av, ai, bv, bi):
    sel = av >= bv
    return (jnp.where(sel, av, bv), jnp.where(sel, ai, bi),
            jnp.where(sel, bv, av), jnp.where(sel, bi, ai))


def _rev(x):
    return jax.lax.rev(x, dimensions=(0,))


def _vsortd(v, i):
    return plsc.sort_key_val(v, i, descending=True)


def _merge16(av, ai, bv, bi):
    hv, hi_, lv, li = _cmpx(av, ai, _rev(bv), _rev(bi))
    hv, hi_ = _vsortd(hv, hi_)
    lv, li = _vsortd(lv, li)
    return hv, hi_, lv, li


def _bmerge32(h0v, h0i, h1v, h1i):
    uv, ui, wv, wi = _cmpx(h0v, h0i, h1v, h1i)
    uv, ui = _vsortd(uv, ui)
    wv, wi = _vsortd(wv, wi)
    return uv, ui, wv, wi


def _merge32(a, b):
    a0v, a0i, a1v, a1i = a
    b0v, b0i, b1v, b1i = b
    h0v, h0i, l1v, l1i = _cmpx(a0v, a0i, _rev(b1v), _rev(b1i))
    h1v, h1i, l0v, l0i = _cmpx(a1v, a1i, _rev(b0v), _rev(b0i))
    s0v, s0i, s1v, s1i = _bmerge32(h0v, h0i, h1v, h1i)
    s2v, s2i, s3v, s3i = _bmerge32(l0v, l0i, l1v, l1i)
    return ((s0v, s0i), (s1v, s1i), (s2v, s2i), (s3v, s3i))


def _sort64(vs, is_):
    s = [_vsortd(vs[k], is_[k]) for k in range(4)]
    a = _merge16(s[0][0], s[0][1], s[1][0], s[1][1])
    b = _merge16(s[2][0], s[2][1], s[3][0], s[3][1])
    return _merge32(a, b)


def _merge_top64(A, B):
    h = []
    for k in range(4):
        bv, bi = B[3 - k]
        hv, hi_, _, _ = _cmpx(A[k][0], A[k][1], _rev(bv), _rev(bi))
        h.append((hv, hi_))
    u0v, u0i, w0v, w0i = _cmpx(h[0][0], h[0][1], h[2][0], h[2][1])
    u1v, u1i, w1v, w1i = _cmpx(h[1][0], h[1][1], h[3][0], h[3][1])
    s0v, s0i, s1v, s1i = _bmerge32(u0v, u0i, u1v, u1i)
    s2v, s2i, s3v, s3i = _bmerge32(w0v, w0i, w1v, w1i)
    return ((s0v, s0i), (s1v, s1i), (s2v, s2i), (s3v, s3i))


def _sc_topk(f_g, M, lo, *, rows, hidden):
    nchunks = hidden // CHUNK
    nc, ns = 2, 16
    nw = nc * ns
    rpw = rows // nw
    mesh = plsc.VectorSubcoreMesh(core_axis_name="c", subcore_axis_name="s",
                                  num_cores=nc, num_subcores=ns)

    @functools.partial(
        pl.kernel, mesh=mesh,
        out_type=(
            jax.ShapeDtypeStruct((rows, hidden), jnp.float32),
            jax.ShapeDtypeStruct((rows, K), jnp.int32),
        ),
        scratch_types=[
            pltpu.VMEM((rpw,), jnp.float32),
            pltpu.VMEM((nchunks,), jnp.float32),
            pltpu.VMEM((CAPC + 16,), jnp.int32),
            pltpu.VMEM((CAPC + 16,), jnp.int32),
            pltpu.VMEM((CAPC, 128), jnp.float32),
            pltpu.VMEM((CAPP + 64,), jnp.float32),
            pltpu.VMEM((CAPP + 64,), jnp.int32),
            pltpu.VMEM((hidden,), jnp.float32),
            pltpu.VMEM((K,), jnp.int32),
            pltpu.SemaphoreType.DMA,
        ],
        compiler_params=pltpu.CompilerParams(needs_layout_passes=False),
    )
    def body(fg_hbm, m_hbm, lo_hbm, sp_hbm, ti_hbm,
             lo_v, m_v, cidx_v, pidx_v, gath_v, vpool, ipool, row_buf, tib,
             sem):
        wid = lax.axis_index("s") * nc + lax.axis_index("c")
        base = wid * rpw
        iota16 = jax.lax.iota(jnp.int32, 16)
        zero16 = jnp.zeros((16,), jnp.float32)

        pltpu.sync_copy(lo_hbm.at[pl.ds(base * 1, rpw)], lo_v)

        @pl.loop(0, hidden // 16)
        def _zero(i):
            row_buf[pl.ds(i * 16, 16)] = zero16

        @pl.loop(0, rpw)
        def _row(rl):
            r = base + rl
            pltpu.sync_copy(m_hbm.at[r], m_v)
            lvec = lo_v[pl.ds((rl // 16) * 16, 16)]
            lsc = jax.lax.reduce_max(
                jnp.where(iota16 == rl % 16, lvec, -1.0), axes=(0,))
            lo_spl = jnp.broadcast_to(lsc, (16,))
            pbase = r * (hidden // 128)
            pad16 = jnp.broadcast_to(pbase, (16,))

            @pl.loop(0, CAPC // 16)
            def _pad(i):
                pidx_v[pl.ds(i * 16, 16)] = pad16

            def scan_body(i, ncand):
                v = m_v[pl.ds(i * 16, 16)]
                msk = (v >= lo_spl) & (ncand < CAPC)
                ids = iota16 + i * 16
                plsc.store_compressed(cidx_v.at[pl.ds(ncand, 16)], ids,
                                      mask=msk)
                plsc.store_compressed(pidx_v.at[pl.ds(ncand, 16)],
                                      pad16 + jax.lax.shift_right_logical(
                                          ids, 3), mask=msk)
                cnt = plsc.all_reduce_population_count(msk)
                return ncand + jax.lax.reduce_max(cnt, axes=(0,))

            ncand = jax.lax.fori_loop(0, nchunks // 16, scan_body, 0,
                                      unroll=4)

            for p in range(CAPC // 128):
                @pl.when(ncand > p * 128)
                def _gather():
                    pltpu.async_copy(
                        fg_hbm.at[pidx_v.at[pl.ds(p * 128, 128)]],
                        gath_v.at[pl.ds(p * 128, 128)], sem).wait()

            def ext_body(j, np_):
                cvec = cidx_v[pl.ds((j // 16) * 16, 16)]
                csc = jax.lax.reduce_max(
                    jnp.where(iota16 == j % 16, cvec, 0), axes=(0,))
                g = gath_v[j, pl.ds((csc % 8) * 16, 16)]
                gidx = jnp.broadcast_to(csc, (16,)) * CHUNK + iota16
                msk = (g >= lo_spl) & (np_ < CAPP)
                plsc.store_compressed(vpool.at[pl.ds(np_, 16)], g, mask=msk)
                plsc.store_compressed(ipool.at[pl.ds(np_, 16)], gidx,
                                      mask=msk)
                cnt = plsc.all_reduce_population_count(msk)
                return np_ + jax.lax.reduce_max(cnt, axes=(0,))

            np_ = jax.lax.fori_loop(0, ncand, ext_body, 0)

            neg1 = jnp.full((16,), -1.0, jnp.float32)
            big = jnp.full((16,), jnp.int32(1 << 24), jnp.int32)
            for t in range(4):
                vpool[pl.ds(np_ + t * 16, 16)] = neg1
                ipool[pl.ds(np_ + t * 16, 16)] = big

            def load64(off):
                vs = [vpool[pl.ds(off + t * 16, 16)] for t in range(4)]
                is_ = [ipool[pl.ds(off + t * 16, 16)] for t in range(4)]
                return _sort64(vs, is_)

            npool = (np_ + 63) // 64

            def mrg_body(q, top):
                return _merge_top64(top, load64(q * 64))

            top = jax.lax.fori_loop(1, npool, mrg_body, load64(0))

            for t in range(4):
                tib[pl.ds(t * 16, 16)] = top[t][1]
                plsc.store_scatter(row_buf, [top[t][1]], top[t][0])
            pltpu.sync_copy(tib, ti_hbm.at[r])
            pltpu.sync_copy(row_buf, sp_hbm.at[r])
            for t in range(4):
                plsc.store_scatter(row_buf, [top[t][1]], zero16)

    return body(f_g, M, lo)


def kernel(x, W, b, threshold):
    batch, seq, in_dim = x.shape
    hidden = W.shape[0]
    rows = batch * seq
    x2d = x.reshape(rows, in_dim)
    b2d = b.reshape(1, hidden)
    th2d = threshold.reshape(1, hidden)
    nh = 4 if rows % (4 * 32) == 0 else 1
    hr = rows // nh
    sps, idxs = [], []
    for i in range(nh):
        xi = jax.lax.slice_in_dim(x2d, i * hr, (i + 1) * hr, axis=0)
        F, M, lo = _encode_tc(xi, W, b2d, th2d, br=min(512, hr), bh=1024)
        f_g = F.reshape(hr * (hidden // 128), 128)
        sp, ix = _sc_topk(f_g, M, lo.reshape(hr), rows=hr, hidden=hidden)
        sps.append(sp)
        idxs.append(ix)
    sparse = jnp.concatenate(sps, axis=0)
    idx = jnp.concatenate(idxs, axis=0)
    return (sparse.reshape(batch, seq, hidden),
            idx.reshape(batch, seq, K))


# speedup vs baseline: 9.6933x; 1.0011x over previous
import functools

import jax
import jax.numpy as jnp
from jax import lax
from jax.experimental import pallas as pl
from jax.experimental.pallas import tpu as pltpu
from jax.experimental.pallas import tpu_sc as plsc

K = 64
CHUNK = 16
CAPC = 256
CAPP = 512


def _mm_kernel(x_ref, w_ref, b_ref, th_ref, f_ref, m_ref, lo_ref, m_scr,
               mprev_scr, *, n_hb, bh, br, nchunks):
    h = pl.program_id(1)
    nct = bh // CHUNK

    pre = jax.lax.dot_general(
        x_ref[...], w_ref[...], (((1,), (1,)), ((), ())),
        preferred_element_type=jnp.float32)
    pre = pre + b_ref[...]
    feat = pre * (pre > th_ref[...]).astype(jnp.float32)
    f_ref[...] = feat
    tmax = jnp.max(feat.reshape(br, nct, CHUNK), axis=2)

    @pl.when(h % 2 == 0)
    def _stash():
        mprev_scr[...] = tmax

    @pl.when(h % 2 == 1)
    def _commit():
        m_scr[:, pl.ds((h // 2) * (2 * nct), 2 * nct)] = jnp.concatenate(
            [mprev_scr[...], tmax], axis=1)

    @pl.when(h == n_hb - 1)
    def _lo_search():
        m_int = jax.lax.bitcast_convert_type(m_scr[...], jnp.int32)

        def body(_, lohi):
            lo, hi = lohi
            mid = lo + (hi - lo) // 2
            cnt = jnp.sum((m_int >= mid[:, None]).astype(jnp.int32), axis=1)
            ok = cnt >= K
            return jnp.where(ok, mid, lo), jnp.where(ok, hi, mid)

        lo0 = jnp.zeros((br,), jnp.int32)
        hi0 = jnp.full((br,), 0x7F800000, jnp.int32)
        lo, _ = jax.lax.fori_loop(0, 31, body, (lo0, hi0))
        m_ref[...] = m_scr[...]
        lo_ref[...] = jax.lax.bitcast_convert_type(lo, jnp.float32)[:, None]


def _encode_tc(x2d, W, b2d, th2d, *, br, bh):
    rows, in_dim = x2d.shape
    hidden = W.shape[0]
    n_rb = rows // br
    n_hb = hidden // bh
    nchunks = hidden // CHUNK
    kern = functools.partial(_mm_kernel, n_hb=n_hb, bh=bh, br=br,
                             nchunks=nchunks)
    return pl.pallas_call(
        kern,
        grid=(n_rb, n_hb),
        in_specs=[
            pl.BlockSpec((br, in_dim), lambda r, h: (r, 0)),
            pl.BlockSpec((bh, in_dim), lambda r, h: (h, 0)),
            pl.BlockSpec((1, bh), lambda r, h: (0, h)),
            pl.BlockSpec((1, bh), lambda r, h: (0, h)),
        ],
        out_specs=(
            pl.BlockSpec((br, bh), lambda r, h: (r, h)),
            pl.BlockSpec((br, nchunks), lambda r, h: (r, 0)),
            pl.BlockSpec((br, 1), lambda r, h: (r, 0)),
        ),
        out_shape=(
            jax.ShapeDtypeStruct((rows, hidden), jnp.float32),
            jax.ShapeDtypeStruct((rows, nchunks), jnp.float32),
            jax.ShapeDtypeStruct((rows, 1), jnp.float32),
        ),
        scratch_shapes=[pltpu.VMEM((br, nchunks), jnp.float32),
                        pltpu.VMEM((br, bh // CHUNK), jnp.float32)],
        compiler_params=pltpu.CompilerParams(
            dimension_semantics=("arbitrary", "arbitrary")),
    )(x2d, W, b2d, th2d)


def _cmpx(av, ai, bv, bi):
    sel = av >= bv
    return (jnp.where(sel, av, bv), jnp.where(sel, ai, bi),
            jnp.where(sel, bv, av), jnp.where(sel, bi, ai))


def _rev(x):
    return jax.lax.rev(x, dimensions=(0,))


def _vsortd(v, i):
    return plsc.sort_key_val(v, i, descending=True)


def _merge16(av, ai, bv, bi):
    hv, hi_, lv, li = _cmpx(av, ai, _rev(bv), _rev(bi))
    hv, hi_ = _vsortd(hv, hi_)
    lv, li = _vsortd(lv, li)
    return hv, hi_, lv, li


def _bmerge32(h0v, h0i, h1v, h1i):
    uv, ui, wv, wi = _cmpx(h0v, h0i, h1v, h1i)
    uv, ui = _vsortd(uv, ui)
    wv, wi = _vsortd(wv, wi)
    return uv, ui, wv, wi


def _merge32(a, b):
    a0v, a0i, a1v, a1i = a
    b0v, b0i, b1v, b1i = b
    h0v, h0i, l1v, l1i = _cmpx(a0v, a0i, _rev(b1v), _rev(b1i))
    h1v, h1i, l0v, l0i = _cmpx(a1v, a1i, _rev(b0v), _rev(b0i))
    s0v, s0i, s1v, s1i = _bmerge32(h0v, h0i, h1v, h1i)
    s2v, s2i, s3v, s3i = _bmerge32(l0v, l0i, l1v, l1i)
    return ((s0v, s0i), (s1v, s1i), (s2v, s2i), (s3v, s3i))


def _sort64(vs, is_):
    s = [_vsortd(vs[k], is_[k]) for k in range(4)]
    a = _merge16(s[0][0], s[0][1], s[1][0], s[1][1])
    b = _merge16(s[2][0], s[2][1], s[3][0], s[3][1])
    return _merge32(a, b)


def _merge_top64(A, B):
    h = []
    for k in range(4):
        bv, bi = B[3 - k]
        hv, hi_, _, _ = _cmpx(A[k][0], A[k][1], _rev(bv), _rev(bi))
        h.append((hv, hi_))
    u0v, u0i, w0v, w0i = _cmpx(h[0][0], h[0][1], h[2][0], h[2][1])
    u1v, u1i, w1v, w1i = _cmpx(h[1][0], h[1][1], h[3][0], h[3][1])
    s0v, s0i, s1v, s1i = _bmerge32(u0v, u0i, u1v, u1i)
    s2v, s2i, s3v, s3i = _bmerge32(w0v, w0i, w1v, w1i)
    return ((s0v, s0i), (s1v, s1i), (s2v, s2i), (s3v, s3i))


def _sc_topk(f_g, M, lo, *, rows, hidden):
    nchunks = hidden // CHUNK
    nc, ns = 2, 16
    nw = nc * ns
    rpw = rows // nw
    mesh = plsc.VectorSubcoreMesh(core_axis_name="c", subcore_axis_name="s",
                                  num_cores=nc, num_subcores=ns)

    @functools.partial(
        pl.kernel, mesh=mesh,
        out_type=(
            jax.ShapeDtypeStruct((rows, hidden), jnp.float32),
            jax.ShapeDtypeStruct((rows * K,), jnp.int32),
        ),
        scratch_types=[
            pltpu.VMEM((rpw,), jnp.float32),
            pltpu.VMEM((nchunks,), jnp.float32),
            pltpu.VMEM((CAPC + 16,), jnp.int32),
            pltpu.VMEM((CAPC + 16,), jnp.int32),
            pltpu.VMEM((CAPC, 128), jnp.float32),
            pltpu.VMEM((CAPP + 64,), jnp.float32),
            pltpu.VMEM((CAPP + 64,), jnp.int32),
            pltpu.VMEM((2 * hidden,), jnp.float32),
            pltpu.VMEM((rpw * K,), jnp.int32),
            pltpu.SemaphoreType.DMA,
            pltpu.SemaphoreType.DMA,
        ],
        compiler_params=pltpu.CompilerParams(needs_layout_passes=False),
    )
    def body(fg_hbm, m_hbm, lo_hbm, sp_hbm, ti_hbm,
             lo_v, m_v, cidx_v, pidx_v, gath_v, vpool, ipool, row_buf, tstage,
             sem, sem_out):
        wid = lax.axis_index("s") * nc + lax.axis_index("c")
        base = wid * rpw
        iota16 = jax.lax.iota(jnp.int32, 16)
        zero16 = jnp.zeros((16,), jnp.float32)

        pltpu.sync_copy(lo_hbm.at[pl.ds(base * 1, rpw)], lo_v)

        @pl.loop(0, 2 * (hidden // 16))
        def _zero(i):
            row_buf[pl.ds(i * 16, 16)] = zero16

        @pl.loop(0, rpw)
        def _row(rl):
            r = base + rl
            boff = (rl % 2) * hidden

            @pl.when(rl >= 2)
            def _drain():
                pltpu.make_async_copy(row_buf.at[pl.ds(boff, hidden)],
                                      sp_hbm.at[r - 2], sem_out).wait()
                bv = jnp.broadcast_to(boff, (16,))
                for t in range(4):
                    iv = tstage[pl.ds((rl - 2) * K + t * 16, 16)]
                    plsc.store_scatter(row_buf, [iv + bv], zero16)
            pltpu.sync_copy(m_hbm.at[r], m_v)
            lvec = lo_v[pl.ds((rl // 16) * 16, 16)]
            lsc = jax.lax.reduce_max(
                jnp.where(iota16 == rl % 16, lvec, -1.0), axes=(0,))
            lo_spl = jnp.broadcast_to(lsc, (16,))
            pbase = r * (hidden // 128)
            pad16 = jnp.broadcast_to(pbase, (16,))

            @pl.loop(0, CAPC // 16)
            def _pad(i):
                pidx_v[pl.ds(i * 16, 16)] = pad16

            def scan_body(i, ncand):
                v = m_v[pl.ds(i * 16, 16)]
                msk = (v >= lo_spl) & (ncand < CAPC)
                ids = iota16 + i * 16
                plsc.store_compressed(cidx_v.at[pl.ds(ncand, 16)], ids,
                                      mask=msk)
                plsc.store_compressed(pidx_v.at[pl.ds(ncand, 16)],
                                      pad16 + jax.lax.shift_right_logical(
                                          ids, 3), mask=msk)
                cnt = plsc.all_reduce_population_count(msk)
                return ncand + jax.lax.reduce_max(cnt, axes=(0,))

            ncand = jax.lax.fori_loop(0, nchunks // 16, scan_body, 0,
                                      unroll=4)

            for p in range(CAPC // 128):
                @pl.when(ncand > p * 128)
                def _gather():
                    pltpu.async_copy(
                        fg_hbm.at[pidx_v.at[pl.ds(p * 128, 128)]],
                        gath_v.at[pl.ds(p * 128, 128)], sem).wait()

            def ext_body(j, np_):
                cvec = cidx_v[pl.ds((j // 16) * 16, 16)]
                csc = jax.lax.reduce_max(
                    jnp.where(iota16 == j % 16, cvec, 0), axes=(0,))
                g = gath_v[j, pl.ds((csc % 8) * 16, 16)]
                gidx = jnp.broadcast_to(csc, (16,)) * CHUNK + iota16
                msk = (g >= lo_spl) & (np_ < CAPP)
                plsc.store_compressed(vpool.at[pl.ds(np_, 16)], g, mask=msk)
                plsc.store_compressed(ipool.at[pl.ds(np_, 16)], gidx,
                                      mask=msk)
                cnt = plsc.all_reduce_population_count(msk)
                return np_ + jax.lax.reduce_max(cnt, axes=(0,))

            np_ = jax.lax.fori_loop(0, ncand, ext_body, 0)

            neg1 = jnp.full((16,), -1.0, jnp.float32)
            big = jnp.full((16,), jnp.int32(1 << 24), jnp.int32)
            for t in range(4):
                vpool[pl.ds(np_ + t * 16, 16)] = neg1
                ipool[pl.ds(np_ + t * 16, 16)] = big

            def load64(off):
                vs = [vpool[pl.ds(off + t * 16, 16)] for t in range(4)]
                is_ = [ipool[pl.ds(off + t * 16, 16)] for t in range(4)]
                return _sort64(vs, is_)

            npool = (np_ + 63) // 64

            def mrg_body(q, top):
                return _merge_top64(top, load64(q * 64))

            top = jax.lax.fori_loop(1, npool, mrg_body, load64(0))

            bv2 = jnp.broadcast_to(boff, (16,))
            for t in range(4):
                tstage[pl.ds(rl * K + t * 16, 16)] = top[t][1]
                plsc.store_scatter(row_buf, [top[t][1] + bv2], top[t][0])
            pltpu.async_copy(row_buf.at[pl.ds(boff, hidden)],
                             sp_hbm.at[r], sem_out)

        for q in (rpw - 2, rpw - 1):
            pltpu.make_async_copy(row_buf.at[pl.ds((q % 2) * hidden, hidden)],
                                  sp_hbm.at[base + q], sem_out).wait()
        pltpu.sync_copy(tstage, ti_hbm.at[pl.ds(base * K, rpw * K)])

    return body(f_g, M, lo)


def kernel(x, W, b, threshold):
    batch, seq, in_dim = x.shape
    hidden = W.shape[0]
    rows = batch * seq
    x2d = x.reshape(rows, in_dim)
    b2d = b.reshape(1, hidden)
    th2d = threshold.reshape(1, hidden)
    nh = 4 if rows % (4 * 32) == 0 else 1
    hr = rows // nh
    sps, idxs = [], []
    for i in range(nh):
        xi = jax.lax.slice_in_dim(x2d, i * hr, (i + 1) * hr, axis=0)
        F, M, lo = _encode_tc(xi, W, b2d, th2d, br=min(512, hr), bh=1024)
        f_g = F.reshape(hr * (hidden // 128), 128)
        sp, ix = _sc_topk(f_g, M, lo.reshape(hr), rows=hr, hidden=hidden)
        sps.append(sp)
        idxs.append(ix.reshape(hr, K))
    sparse = jnp.concatenate(sps, axis=0)
    idx = jnp.concatenate(idxs, axis=0)
    return (sparse.reshape(batch, seq, hidden),
            idx.reshape(batch, seq, K))


# speedup vs baseline: 10.1284x; 1.0449x over previous
import functools

import jax
import jax.numpy as jnp
from jax import lax
from jax.experimental import pallas as pl
from jax.experimental.pallas import tpu as pltpu
from jax.experimental.pallas import tpu_sc as plsc

K = 64
CHUNK = 16
CAPC = 256
CAPP = 512


def _mm_kernel(x_ref, w_ref, b_ref, th_ref, f_ref, m_ref, lo_ref, m_scr,
               mprev_scr, *, n_hb, bh, br, nchunks):
    h = pl.program_id(1)
    nct = bh // CHUNK

    pre = jax.lax.dot_general(
        x_ref[...], w_ref[...], (((1,), (1,)), ((), ())),
        preferred_element_type=jnp.float32)
    pre = pre + b_ref[...]
    feat = pre * (pre > th_ref[...]).astype(jnp.float32)
    f_ref[...] = feat
    tmax = jnp.max(feat.reshape(br, nct, CHUNK), axis=2)

    @pl.when(h % 2 == 0)
    def _stash():
        mprev_scr[...] = tmax

    @pl.when(h % 2 == 1)
    def _commit():
        m_scr[:, pl.ds((h // 2) * (2 * nct), 2 * nct)] = jnp.concatenate(
            [mprev_scr[...], tmax], axis=1)

    @pl.when(h == n_hb - 1)
    def _lo_search():
        m_int = jax.lax.bitcast_convert_type(m_scr[...], jnp.int32)

        def body(_, lohi):
            lo, hi = lohi
            mid = lo + (hi - lo) // 2
            cnt = jnp.sum((m_int >= mid[:, None]).astype(jnp.int32), axis=1)
            ok = cnt >= K
            return jnp.where(ok, mid, lo), jnp.where(ok, hi, mid)

        lo0 = jnp.zeros((br,), jnp.int32)
        hi0 = jnp.full((br,), 0x7F800000, jnp.int32)
        lo, _ = jax.lax.fori_loop(0, 31, body, (lo0, hi0))
        m_ref[...] = m_scr[...]
        lo_ref[...] = jax.lax.bitcast_convert_type(lo, jnp.float32)[:, None]


def _encode_tc(x2d, W, b2d, th2d, *, br, bh):
    rows, in_dim = x2d.shape
    hidden = W.shape[0]
    n_rb = rows // br
    n_hb = hidden // bh
    nchunks = hidden // CHUNK
    kern = functools.partial(_mm_kernel, n_hb=n_hb, bh=bh, br=br,
                             nchunks=nchunks)
    return pl.pallas_call(
        kern,
        grid=(n_rb, n_hb),
        in_specs=[
            pl.BlockSpec((br, in_dim), lambda r, h: (r, 0)),
            pl.BlockSpec((bh, in_dim), lambda r, h: (h, 0)),
            pl.BlockSpec((1, bh), lambda r, h: (0, h)),
            pl.BlockSpec((1, bh), lambda r, h: (0, h)),
        ],
        out_specs=(
            pl.BlockSpec((br, bh), lambda r, h: (r, h)),
            pl.BlockSpec((br, nchunks), lambda r, h: (r, 0)),
            pl.BlockSpec((br, 1), lambda r, h: (r, 0)),
        ),
        out_shape=(
            jax.ShapeDtypeStruct((rows, hidden), jnp.float32),
            jax.ShapeDtypeStruct((rows, nchunks), jnp.float32),
            jax.ShapeDtypeStruct((rows, 1), jnp.float32),
        ),
        scratch_shapes=[pltpu.VMEM((br, nchunks), jnp.float32),
                        pltpu.VMEM((br, bh // CHUNK), jnp.float32)],
        compiler_params=pltpu.CompilerParams(
            dimension_semantics=("arbitrary", "arbitrary")),
    )(x2d, W, b2d, th2d)


def _cmpx(av, ai, bv, bi):
    sel = av >= bv
    return (jnp.where(sel, av, bv), jnp.where(sel, ai, bi),
            jnp.where(sel, bv, av), jnp.where(sel, bi, ai))


def _rev(x):
    return jax.lax.rev(x, dimensions=(0,))


def _vsortd(v, i):
    return plsc.sort_key_val(v, i, descending=True)


def _merge16(av, ai, bv, bi):
    hv, hi_, lv, li = _cmpx(av, ai, _rev(bv), _rev(bi))
    hv, hi_ = _vsortd(hv, hi_)
    lv, li = _vsortd(lv, li)
    return hv, hi_, lv, li


def _bmerge32(h0v, h0i, h1v, h1i):
    uv, ui, wv, wi = _cmpx(h0v, h0i, h1v, h1i)
    uv, ui = _vsortd(uv, ui)
    wv, wi = _vsortd(wv, wi)
    return uv, ui, wv, wi


def _merge32(a, b):
    a0v, a0i, a1v, a1i = a
    b0v, b0i, b1v, b1i = b
    h0v, h0i, l1v, l1i = _cmpx(a0v, a0i, _rev(b1v), _rev(b1i))
    h1v, h1i, l0v, l0i = _cmpx(a1v, a1i, _rev(b0v), _rev(b0i))
    s0v, s0i, s1v, s1i = _bmerge32(h0v, h0i, h1v, h1i)
    s2v, s2i, s3v, s3i = _bmerge32(l0v, l0i, l1v, l1i)
    return ((s0v, s0i), (s1v, s1i), (s2v, s2i), (s3v, s3i))


def _sort64(vs, is_):
    s = [_vsortd(vs[k], is_[k]) for k in range(4)]
    a = _merge16(s[0][0], s[0][1], s[1][0], s[1][1])
    b = _merge16(s[2][0], s[2][1], s[3][0], s[3][1])
    return _merge32(a, b)


def _merge_top64(A, B):
    h = []
    for k in range(4):
        bv, bi = B[3 - k]
        hv, hi_, _, _ = _cmpx(A[k][0], A[k][1], _rev(bv), _rev(bi))
        h.append((hv, hi_))
    u0v, u0i, w0v, w0i = _cmpx(h[0][0], h[0][1], h[2][0], h[2][1])
    u1v, u1i, w1v, w1i = _cmpx(h[1][0], h[1][1], h[3][0], h[3][1])
    s0v, s0i, s1v, s1i = _bmerge32(u0v, u0i, u1v, u1i)
    s2v, s2i, s3v, s3i = _bmerge32(w0v, w0i, w1v, w1i)
    return ((s0v, s0i), (s1v, s1i), (s2v, s2i), (s3v, s3i))


def _sc_topk(f_g, M, lo, *, rows, hidden):
    nchunks = hidden // CHUNK
    nc, ns = 2, 16
    nw = nc * ns
    rpw = rows // nw
    mesh = plsc.VectorSubcoreMesh(core_axis_name="c", subcore_axis_name="s",
                                  num_cores=nc, num_subcores=ns)

    @functools.partial(
        pl.kernel, mesh=mesh,
        out_type=(
            jax.ShapeDtypeStruct((rows, hidden), jnp.float32),
            jax.ShapeDtypeStruct((rows * K,), jnp.int32),
        ),
        scratch_types=[
            pltpu.VMEM((rpw,), jnp.float32),
            pltpu.VMEM((nchunks,), jnp.float32),
            pltpu.VMEM((CAPC + 16,), jnp.int32),
            pltpu.VMEM((CAPC + 16,), jnp.int32),
            pltpu.VMEM((CAPC, 128), jnp.float32),
            pltpu.VMEM((CAPP + 64,), jnp.float32),
            pltpu.VMEM((CAPP + 64,), jnp.int32),
            pltpu.VMEM((2 * hidden,), jnp.float32),
            pltpu.VMEM((rpw * K,), jnp.int32),
            pltpu.SemaphoreType.DMA,
            pltpu.SemaphoreType.DMA,
        ],
        compiler_params=pltpu.CompilerParams(needs_layout_passes=False),
    )
    def body(fg_hbm, m_hbm, lo_hbm, sp_hbm, ti_hbm,
             lo_v, m_v, cidx_v, pidx_v, gath_v, vpool, ipool, row_buf, tstage,
             sem, sem_out):
        wid = lax.axis_index("s") * nc + lax.axis_index("c")
        base = wid * rpw
        iota16 = jax.lax.iota(jnp.int32, 16)
        zero16 = jnp.zeros((16,), jnp.float32)

        pltpu.sync_copy(lo_hbm.at[pl.ds(base * 1, rpw)], lo_v)

        @pl.loop(0, 2 * (hidden // 16))
        def _zero(i):
            row_buf[pl.ds(i * 16, 16)] = zero16

        @pl.loop(0, rpw)
        def _row(rl):
            r = base + rl
            boff = (rl % 2) * hidden

            @pl.when(rl >= 2)
            def _drain():
                pltpu.make_async_copy(row_buf.at[pl.ds(boff, hidden)],
                                      sp_hbm.at[r - 2], sem_out).wait()
                bv = jnp.broadcast_to(boff, (16,))
                for t in range(4):
                    iv = tstage[pl.ds((rl - 2) * K + t * 16, 16)]
                    plsc.store_scatter(row_buf, [iv + bv], zero16)
            pltpu.sync_copy(m_hbm.at[r], m_v)
            lvec = lo_v[pl.ds((rl // 16) * 16, 16)]
            lsc = jax.lax.reduce_max(
                jnp.where(iota16 == rl % 16, lvec, -1.0), axes=(0,))
            lo_spl = jnp.broadcast_to(lsc, (16,))
            pbase = r * (hidden // 128)
            pad16 = jnp.broadcast_to(pbase, (16,))

            @pl.loop(0, CAPC // 16)
            def _pad(i):
                pidx_v[pl.ds(i * 16, 16)] = pad16

            def scan_body(i, ncand):
                v = m_v[pl.ds(i * 16, 16)]
                msk = (v >= lo_spl) & (ncand < CAPC)
                ids = iota16 + i * 16
                plsc.store_compressed(cidx_v.at[pl.ds(ncand, 16)], ids,
                                      mask=msk)
                plsc.store_compressed(pidx_v.at[pl.ds(ncand, 16)],
                                      pad16 + jax.lax.shift_right_logical(
                                          ids, 3), mask=msk)
                cnt = plsc.all_reduce_population_count(msk)
                return ncand + jax.lax.reduce_max(cnt, axes=(0,))

            ncand = jax.lax.fori_loop(0, nchunks // 16, scan_body, 0,
                                      unroll=4)

            for p in range(CAPC // 128):
                @pl.when(ncand > p * 128)
                def _gather():
                    pltpu.async_copy(
                        fg_hbm.at[pidx_v.at[pl.ds(p * 128, 128)]],
                        gath_v.at[pl.ds(p * 128, 128)], sem).wait()

            def ext_body(j, np_):
                cvec = cidx_v[pl.ds((j // 16) * 16, 16)]
                csc = jax.lax.reduce_max(
                    jnp.where(iota16 == j % 16, cvec, 0), axes=(0,))
                g = gath_v[j, pl.ds((csc % 8) * 16, 16)]
                gidx = jnp.broadcast_to(csc, (16,)) * CHUNK + iota16
                msk = (g >= lo_spl) & (np_ < CAPP)
                plsc.store_compressed(vpool.at[pl.ds(np_, 16)], g, mask=msk)
                plsc.store_compressed(ipool.at[pl.ds(np_, 16)], gidx,
                                      mask=msk)
                cnt = plsc.all_reduce_population_count(msk)
                return np_ + jax.lax.reduce_max(cnt, axes=(0,))

            np_ = jax.lax.fori_loop(0, ncand, ext_body, 0)

            neg1 = jnp.full((16,), -1.0, jnp.float32)
            big = jnp.full((16,), jnp.int32(1 << 24), jnp.int32)
            for t in range(4):
                vpool[pl.ds(np_ + t * 16, 16)] = neg1
                ipool[pl.ds(np_ + t * 16, 16)] = big

            def load64(off):
                vs = [vpool[pl.ds(off + t * 16, 16)] for t in range(4)]
                is_ = [ipool[pl.ds(off + t * 16, 16)] for t in range(4)]
                return _sort64(vs, is_)

            npool = (np_ + 63) // 64

            def mrg_body(q, top):
                return _merge_top64(top, load64(q * 64))

            top = jax.lax.fori_loop(1, npool, mrg_body, load64(0))

            bv2 = jnp.broadcast_to(boff, (16,))
            for t in range(4):
                tstage[pl.ds(rl * K + t * 16, 16)] = top[t][1]
                plsc.store_scatter(row_buf, [top[t][1] + bv2], top[t][0])
            pltpu.async_copy(row_buf.at[pl.ds(boff, hidden)],
                             sp_hbm.at[r], sem_out)

        for q in (rpw - 2, rpw - 1):
            pltpu.make_async_copy(row_buf.at[pl.ds((q % 2) * hidden, hidden)],
                                  sp_hbm.at[base + q], sem_out).wait()
        pltpu.sync_copy(tstage, ti_hbm.at[pl.ds(base * K, rpw * K)])

    return body(f_g, M, lo)


def kernel(x, W, b, threshold):
    batch, seq, in_dim = x.shape
    hidden = W.shape[0]
    rows = batch * seq
    x2d = x.reshape(rows, in_dim)
    b2d = b.reshape(1, hidden)
    th2d = threshold.reshape(1, hidden)
    nh = 8 if rows % (8 * 64) == 0 else 1
    hr = rows // nh
    sps, idxs = [], []
    for i in range(nh):
        xi = jax.lax.slice_in_dim(x2d, i * hr, (i + 1) * hr, axis=0)
        F, M, lo = _encode_tc(xi, W, b2d, th2d, br=min(512, hr), bh=1024)
        f_g = F.reshape(hr * (hidden // 128), 128)
        sp, ix = _sc_topk(f_g, M, lo.reshape(hr), rows=hr, hidden=hidden)
        sps.append(sp)
        idxs.append(ix.reshape(hr, K))
    sparse = jnp.concatenate(sps, axis=0)
    idx = jnp.concatenate(idxs, axis=0)
    return (sparse.reshape(batch, seq, hidden),
            idx.reshape(batch, seq, K))
